# baseline (speedup 1.0000x reference)
import jax
import jax.numpy as jnp
from jax.experimental import pallas as pl
from jax.experimental.pallas import tpu as pltpu

FEATURE_DIM = 2048
NUM_EXPERT = 16
TOKENS = 8192
BLOCK_T = 1024


def _router_block(x_ref, w_ref, b_ref, o_ref):
    x = x_ref[...]
    w = w_ref[...]
    logits = jax.lax.dot_general(
        x, w, (((1,), (1,)), ((), ())),
        preferred_element_type=jnp.float32,
    ) + b_ref[...]
    m = jnp.max(logits, axis=-1, keepdims=True)
    e = jnp.exp(logits - m)
    s = jnp.sum(e, axis=-1, keepdims=True)
    o_ref[...] = e / s


def kernel(x, W, b):
    b2 = b.reshape(1, NUM_EXPERT)
    grid = TOKENS // BLOCK_T
    return pl.pallas_call(
        _router_block,
        grid=(grid,),
        in_specs=[
            pl.BlockSpec((BLOCK_T, FEATURE_DIM), lambda i: (i, 0)),
            pl.BlockSpec((NUM_EXPERT, FEATURE_DIM), lambda i: (0, 0)),
            pl.BlockSpec((1, NUM_EXPERT), lambda i: (0, 0)),
        ],
        out_specs=pl.BlockSpec((BLOCK_T, NUM_EXPERT), lambda i: (i, 0)),
        out_shape=jax.ShapeDtypeStruct((TOKENS, NUM_EXPERT), jnp.float32),
    )(x, W, b2)
